# speedup vs baseline: 2.4408x; 2.4408x over previous
import functools

import jax
import jax.numpy as jnp
from jax import lax
from jax.experimental import pallas as pl
from jax.experimental.pallas import tpu as pltpu
from jax.experimental.pallas import tpu_sc as plsc

_EMB = 128
_N_TOKENS = 4 * 8192
_NUM_CORES = 2
_NUM_SUBCORES = 16
_NUM_WORKERS = _NUM_CORES * _NUM_SUBCORES
_ROWS_PER_WORKER = _N_TOKENS // _NUM_WORKERS
_CHUNK = 128
_NUM_CHUNKS = _ROWS_PER_WORKER // _CHUNK
_LANES = 16


@functools.partial(
    pl.kernel,
    out_type=jax.ShapeDtypeStruct((_N_TOKENS, _EMB), jnp.float32),
    mesh=plsc.VectorSubcoreMesh(
        core_axis_name="c",
        subcore_axis_name="s",
        num_cores=_NUM_CORES,
        num_subcores=_NUM_SUBCORES,
    ),
    scratch_types=[
        pltpu.VMEM((_CHUNK,), jnp.int32),
        pltpu.VMEM((_CHUNK,), jnp.int32),
        pltpu.VMEM((_CHUNK, _EMB), jnp.float32),
        pltpu.VMEM((_CHUNK, _EMB), jnp.float32),
        pltpu.SemaphoreType.DMA,
        pltpu.SemaphoreType.DMA,
    ],
)
def _embed_sc(tok_ids_hbm, pos_ids_hbm, tok_tab_hbm, pos_tab_hbm, out_hbm,
              idx_t, idx_p, rows_t, rows_p, sem_t, sem_p):
    wid = lax.axis_index("s") * _NUM_CORES + lax.axis_index("c")
    base = wid * _ROWS_PER_WORKER

    def chunk_body(ci, carry):
        off = base + ci * _CHUNK
        pltpu.sync_copy(tok_ids_hbm.at[pl.ds(off, _CHUNK)], idx_t)
        pltpu.sync_copy(pos_ids_hbm.at[pl.ds(off, _CHUNK)], idx_p)
        cp_t = pltpu.async_copy(tok_tab_hbm.at[idx_t], rows_t, sem_t)
        cp_p = pltpu.async_copy(pos_tab_hbm.at[idx_p], rows_p, sem_p)
        cp_t.wait()
        cp_p.wait()

        def row_body(r, c):
            for j in range(_EMB // _LANES):
                sl = pl.ds(j * _LANES, _LANES)
                plsc.addupdate(rows_t.at[r, sl], rows_p[r, sl])
            return c

        lax.fori_loop(0, _CHUNK, row_body, 0, unroll=2)
        pltpu.sync_copy(rows_t, out_hbm.at[pl.ds(off, _CHUNK)])
        return carry

    lax.fori_loop(0, _NUM_CHUNKS, chunk_body, 0)


def kernel(token_ids, position_ids, tok_table, pos_table):
    batch, seq_len = token_ids.shape
    tok_flat = token_ids.reshape(-1).astype(jnp.int32)
    pos_flat = position_ids.reshape(-1).astype(jnp.int32)
    out = _embed_sc(tok_flat, pos_flat, tok_table, pos_table)
    return out.reshape(batch, seq_len, _EMB)


# speedup vs baseline: 3.4325x; 1.4063x over previous
import functools

import jax
import jax.numpy as jnp
from jax import lax
from jax.experimental import pallas as pl
from jax.experimental.pallas import tpu as pltpu
from jax.experimental.pallas import tpu_sc as plsc

_EMB = 128
_N_TOKENS = 4 * 8192
_NUM_CORES = 2
_NUM_SUBCORES = 16
_NUM_WORKERS = _NUM_CORES * _NUM_SUBCORES
_ROWS_PER_WORKER = _N_TOKENS // _NUM_WORKERS
_CHUNK = 128
_NUM_CHUNKS = _ROWS_PER_WORKER // _CHUNK
_LANES = 16
_NBUF = 3


@functools.partial(
    pl.kernel,
    out_type=jax.ShapeDtypeStruct((_N_TOKENS, _EMB), jnp.float32),
    mesh=plsc.VectorSubcoreMesh(
        core_axis_name="c",
        subcore_axis_name="s",
        num_cores=_NUM_CORES,
        num_subcores=_NUM_SUBCORES,
    ),
    scratch_types=[
        pltpu.VMEM((_NUM_CHUNKS, _CHUNK), jnp.int32),
        pltpu.VMEM((_NUM_CHUNKS, _CHUNK), jnp.int32),
    ]
    + [pltpu.VMEM((_CHUNK, _EMB), jnp.float32) for _ in range(2 * _NBUF)]
    + [pltpu.SemaphoreType.DMA for _ in range(3 * _NBUF)],
)
def _embed_sc(tok_ids_hbm, pos_ids_hbm, tok_tab_hbm, pos_tab_hbm, out_hbm,
              idx_t, idx_p,
              rt0, rt1, rt2, rp0, rp1, rp2,
              sgt0, sgt1, sgt2, sgp0, sgp1, sgp2, swb0, swb1, swb2):
    rt = (rt0, rt1, rt2)
    rp = (rp0, rp1, rp2)
    sgt = (sgt0, sgt1, sgt2)
    sgp = (sgp0, sgp1, sgp2)
    swb = (swb0, swb1, swb2)

    wid = lax.axis_index("s") * _NUM_CORES + lax.axis_index("c")
    blk = wid * _NUM_CHUNKS

    pltpu.sync_copy(tok_ids_hbm.at[pl.ds(blk, _NUM_CHUNKS)], idx_t)
    pltpu.sync_copy(pos_ids_hbm.at[pl.ds(blk, _NUM_CHUNKS)], idx_p)

    gt = [None] * _NBUF
    gp = [None] * _NBUF
    wb = [None] * _NBUF

    def fire(ci):
        b = ci % _NBUF
        gt[b] = pltpu.async_copy(tok_tab_hbm.at[idx_t.at[ci]], rt[b], sgt[b])
        gp[b] = pltpu.async_copy(pos_tab_hbm.at[idx_p.at[ci]], rp[b], sgp[b])

    for ci in range(min(_NBUF - 1, _NUM_CHUNKS)):
        fire(ci)

    for ci in range(_NUM_CHUNKS):
        b = ci % _NBUF
        gt[b].wait()
        gp[b].wait()

        rtb, rpb = rt[b], rp[b]

        def row_body(r, c):
            for j in range(_EMB // _LANES):
                sl = pl.ds(j * _LANES, _LANES)
                plsc.addupdate(rtb.at[r, sl], rpb[r, sl])
            return c

        lax.fori_loop(0, _CHUNK, row_body, 0, unroll=2)

        off = (blk + ci) * _CHUNK
        wb[b] = pltpu.async_copy(rt[b], out_hbm.at[pl.ds(off, _CHUNK)], swb[b])

        nxt = ci + _NBUF - 1
        if nxt < _NUM_CHUNKS:
            if nxt >= _NBUF:
                wb[nxt % _NBUF].wait()
            fire(nxt)

    for ci in range(max(0, _NUM_CHUNKS - _NBUF), _NUM_CHUNKS):
        wb[ci % _NBUF].wait()


def kernel(token_ids, position_ids, tok_table, pos_table):
    batch, seq_len = token_ids.shape
    tok2 = token_ids.reshape(-1, _CHUNK).astype(jnp.int32)
    pos2 = position_ids.reshape(-1, _CHUNK).astype(jnp.int32)
    out = _embed_sc(tok2, pos2, tok_table, pos_table)
    return out.reshape(batch, seq_len, _EMB)
